# baseline (speedup 1.0000x reference)
import jax
import jax.numpy as jnp
from jax.experimental import pallas as pl
from jax.experimental.pallas import tpu as pltpu
from jax.experimental.pallas import tpu_sc as plsc

B, S, H = 2, 4096, 1024
E, TOPK = 8, 2
INTER = 4096
MOE_INTER = INTER // TOPK
T = B * S

TT_R = 2048
RTILE = 256
NTILES = T * TOPK // RTILE + E
NP = NTILES * RTILE
TT_S = 2048
MM_S = 512
TT_C = 1024
W_SC = 128
HH = H // 2


def _router_kernel(x_ref, w9_ref, logits_ref, idx_ref, w2_ref, sig_ref):
    x = x_ref[...]
    w9 = w9_ref[...]
    l9 = jax.lax.dot_general(x, w9, (((1,), (1,)), ((), ())),
                             preferred_element_type=jnp.float32)
    logits = l9[:, :E]
    logits_ref[...] = logits
    sig_ref[...] = jax.nn.sigmoid(l9[:, E:E + 1])
    m = jnp.max(logits, axis=1, keepdims=True)
    p = jnp.exp(logits - m)
    p = p / jnp.sum(p, axis=1, keepdims=True)
    w1 = jnp.max(p, axis=1, keepdims=True)
    i1 = jnp.argmax(p, axis=1)[:, None]
    cols = jax.lax.broadcasted_iota(jnp.int32, (TT_R, E), 1)
    p2 = jnp.where(cols == i1, -jnp.inf, p)
    w2 = jnp.max(p2, axis=1, keepdims=True)
    i2 = jnp.argmax(p2, axis=1)[:, None]
    idx_ref[...] = jnp.concatenate([i1, i2], axis=1).astype(jnp.int32)
    w2_ref[...] = jnp.concatenate([w1, w2], axis=1)


def _router(x32, w9):
    return pl.pallas_call(
        _router_kernel,
        grid=(T // TT_R,),
        in_specs=[
            pl.BlockSpec((TT_R, H), lambda t: (t, 0)),
            pl.BlockSpec((E + 1, H), lambda t: (0, 0)),
        ],
        out_specs=[
            pl.BlockSpec((TT_R, E), lambda t: (t, 0)),
            pl.BlockSpec((TT_R, TOPK), lambda t: (t, 0)),
            pl.BlockSpec((TT_R, TOPK), lambda t: (t, 0)),
            pl.BlockSpec((TT_R, 1), lambda t: (t, 0)),
        ],
        out_shape=[
            jax.ShapeDtypeStruct((T, E), jnp.float32),
            jax.ShapeDtypeStruct((T, TOPK), jnp.int32),
            jax.ShapeDtypeStruct((T, TOPK), jnp.float32),
            jax.ShapeDtypeStruct((T, 1), jnp.float32),
        ],
    )(x32, w9)


def _meta_kernel(ids_ref, pos_ref, te_ref):
    ids = ids_ref[...]
    r = jax.lax.broadcasted_iota(jnp.int32, (128, 128), 0)
    c = jax.lax.broadcasted_iota(jnp.int32, (128, 128), 1)
    tincl = (r <= c).astype(jnp.bfloat16)
    aexcl = (c < r).astype(jnp.bfloat16)

    cnts = [jnp.sum((ids == e).astype(jnp.int32)) for e in range(E)]
    goff, tstart, ptiles = [], [], []
    off = jnp.int32(0)
    toff = jnp.int32(0)
    for e in range(E):
        goff.append(off)
        tstart.append(toff)
        pt = (cnts[e] + (RTILE - 1)) // RTILE
        ptiles.append(pt)
        off = off + pt * RTILE
        toff = toff + pt

    pos = jnp.zeros((128, 128), jnp.float32)
    for e in range(E):
        me = ids == e
        mef = me.astype(jnp.bfloat16)
        rowcs = jax.lax.dot_general(mef, tincl, (((1,), (0,)), ((), ())),
                                    preferred_element_type=jnp.float32)
        rowsum = rowcs[:, 127:128].astype(jnp.bfloat16)
        roff = jax.lax.dot_general(aexcl, rowsum, (((1,), (0,)), ((), ())),
                                   preferred_element_type=jnp.float32)
        rank1 = rowcs + roff
        pos = pos + jnp.where(me, rank1 - 1.0 + goff[e].astype(jnp.float32), 0.0)
    pos_ref[...] = pos.astype(jnp.int32)

    ti = jax.lax.broadcasted_iota(jnp.int32, (1, 128), 1)
    te = jnp.zeros((1, 128), jnp.int32)
    for e in range(E):
        te = te + (ti >= tstart[e] + ptiles[e]).astype(jnp.int32)
    te_ref[...] = jnp.minimum(te, E - 1)


def _meta(ids128):
    return pl.pallas_call(
        _meta_kernel,
        out_shape=[
            jax.ShapeDtypeStruct((128, 128), jnp.int32),
            jax.ShapeDtypeStruct((1, 128), jnp.int32),
        ],
    )(ids128)


HW = HH // 2


def _half_row_indices(pos2):
    return (2 * pos2[:, :, None] + jnp.arange(2, dtype=jnp.int32)[None, None, :]
            ).reshape(TOPK, 2 * T)


def _as_i32_rows(a, nrows):
    return jax.lax.bitcast_convert_type(a.reshape(nrows, HW, 2), jnp.int32)


def _from_i32_rows(a, nrows, hdim):
    return jax.lax.bitcast_convert_type(a, jnp.bfloat16).reshape(nrows, hdim)


def _sc_scatter(xbf, pos2):
    mesh = plsc.VectorSubcoreMesh(core_axis_name="core", subcore_axis_name="subcore")
    x2 = _as_i32_rows(xbf, 2 * T)
    idx = _half_row_indices(pos2)
    nwin = 2 * T // W_SC

    @pl.kernel(out_type=jax.ShapeDtypeStruct((2 * NP, HW), jnp.int32), mesh=mesh)
    def k(x_hbm, p_hbm, o_hbm):
        def body(x_vmem, p_vmem):
            pltpu.sync_copy(x_vmem, o_hbm.at[p_vmem.at[0]])

        pltpu.emit_pipeline(
            body,
            grid=(TOPK, nwin),
            in_specs=[
                pl.BlockSpec((W_SC, HW), lambda kk, i: (i, 0)),
                pl.BlockSpec((1, W_SC), lambda kk, i: (kk, i)),
            ],
            out_specs=[],
            core_axis_name=("core", "subcore"),
            dimension_semantics=(pltpu.PARALLEL, pltpu.PARALLEL),
        )(x_hbm, p_hbm)

    return _from_i32_rows(k(x2, idx), NP, H)


def _sc_gather(ys, pos2):
    mesh = plsc.VectorSubcoreMesh(core_axis_name="core", subcore_axis_name="subcore")
    ys2 = _as_i32_rows(ys, 2 * NP)
    idx = _half_row_indices(pos2)
    nwin = 2 * T // W_SC

    @pl.kernel(out_type=jax.ShapeDtypeStruct((TOPK * 2 * T, HW), jnp.int32),
               mesh=mesh)
    def k(ys_hbm, p_hbm, o_hbm):
        def body(p_vmem, o_vmem):
            pltpu.sync_copy(ys_hbm.at[p_vmem.at[0]], o_vmem)

        pltpu.emit_pipeline(
            body,
            grid=(TOPK, nwin),
            in_specs=[pl.BlockSpec((1, W_SC), lambda kk, i: (kk, i))],
            out_specs=[pl.BlockSpec((W_SC, HW), lambda kk, i: (kk * nwin + i, 0))],
            core_axis_name=("core", "subcore"),
            dimension_semantics=(pltpu.PARALLEL, pltpu.PARALLEL),
        )(p_hbm, o_hbm)

    return _from_i32_rows(k(ys2, idx), TOPK * T, H)


def _group_kernel(te_ref, xs_ref, wg_ref, wu_ref, wd_ref, ys_ref):
    del te_ref
    x = xs_ref[...]
    g = jax.lax.dot_general(x, wg_ref[0], (((1,), (1,)), ((), ())),
                            preferred_element_type=jnp.float32)
    u = jax.lax.dot_general(x, wu_ref[0], (((1,), (1,)), ((), ())),
                            preferred_element_type=jnp.float32)
    h = (g * jax.nn.sigmoid(g) * u).astype(jnp.bfloat16)
    y = jax.lax.dot_general(h, wd_ref[0], (((1,), (1,)), ((), ())),
                            preferred_element_type=jnp.float32)
    ys_ref[...] = y.astype(jnp.bfloat16)


def _grouped(te_arr, xs, wg, wu, wd):
    grid_spec = pltpu.PrefetchScalarGridSpec(
        num_scalar_prefetch=1,
        grid=(NTILES,),
        in_specs=[
            pl.BlockSpec((RTILE, H), lambda i, te: (i, 0)),
            pl.BlockSpec((1, MOE_INTER, H), lambda i, te: (te[i], 0, 0)),
            pl.BlockSpec((1, MOE_INTER, H), lambda i, te: (te[i], 0, 0)),
            pl.BlockSpec((1, H, MOE_INTER), lambda i, te: (te[i], 0, 0)),
        ],
        out_specs=pl.BlockSpec((RTILE, H), lambda i, te: (i, 0)),
    )
    return pl.pallas_call(
        _group_kernel,
        grid_spec=grid_spec,
        out_shape=jax.ShapeDtypeStruct((NP, H), jnp.bfloat16),
    )(te_arr, xs, wg, wu, wd)


def _shared_kernel(x_ref, wsg_ref, wsu_ref, wsd_ref, out_ref):
    m = pl.program_id(1)

    @pl.when(m == 0)
    def _():
        out_ref[...] = jnp.zeros_like(out_ref)

    x = x_ref[...]
    g = jax.lax.dot_general(x, wsg_ref[...], (((1,), (1,)), ((), ())),
                            preferred_element_type=jnp.float32)
    u = jax.lax.dot_general(x, wsu_ref[...], (((1,), (1,)), ((), ())),
                            preferred_element_type=jnp.float32)
    h = (g * jax.nn.sigmoid(g) * u).astype(jnp.bfloat16)
    out_ref[...] += jax.lax.dot_general(h, wsd_ref[...], (((1,), (1,)), ((), ())),
                                        preferred_element_type=jnp.float32)


def _shared(xbf, wsg, wsu, wsd):
    return pl.pallas_call(
        _shared_kernel,
        grid=(T // TT_S, INTER // MM_S),
        in_specs=[
            pl.BlockSpec((TT_S, H), lambda t, m: (t, 0)),
            pl.BlockSpec((MM_S, H), lambda t, m: (m, 0)),
            pl.BlockSpec((MM_S, H), lambda t, m: (m, 0)),
            pl.BlockSpec((H, MM_S), lambda t, m: (0, m)),
        ],
        out_specs=pl.BlockSpec((TT_S, H), lambda t, m: (t, 0)),
        out_shape=jax.ShapeDtypeStruct((T, H), jnp.float32),
    )(xbf, wsg, wsu, wsd)


def _combine_kernel(sh_ref, y0_ref, y1_ref, w_ref, sig_ref, out_ref):
    w = w_ref[...]
    out_ref[...] = (sig_ref[...] * sh_ref[...]
                    + w[:, 0:1] * y0_ref[...].astype(jnp.float32)
                    + w[:, 1:2] * y1_ref[...].astype(jnp.float32))


def _combine(shared_raw, ycomb, w2, sig):
    return pl.pallas_call(
        _combine_kernel,
        grid=(T // TT_C,),
        in_specs=[
            pl.BlockSpec((TT_C, H), lambda t: (t, 0)),
            pl.BlockSpec((TT_C, H), lambda t: (t, 0)),
            pl.BlockSpec((TT_C, H), lambda t: (T // TT_C + t, 0)),
            pl.BlockSpec((TT_C, TOPK), lambda t: (t, 0)),
            pl.BlockSpec((TT_C, 1), lambda t: (t, 0)),
        ],
        out_specs=pl.BlockSpec((TT_C, H), lambda t: (t, 0)),
        out_shape=jax.ShapeDtypeStruct((T, H), jnp.float32),
    )(shared_raw, ycomb, ycomb, w2, sig)


def kernel(hidden_states, gate_W, expert_gate_W, expert_up_W, expert_down_W,
           shared_gate_W, shared_up_W, shared_down_W, shared_expert_gate_W):
    x32 = hidden_states.reshape(T, H)
    xbf = x32.astype(jnp.bfloat16)
    bf = jnp.bfloat16

    w9 = jnp.concatenate([gate_W, shared_expert_gate_W], axis=0)
    logits, idx2, w2, sig = _router(x32, w9)

    ids128 = idx2.T.reshape(128, 128)
    pos128, te = _meta(ids128)
    pos2 = pos128.reshape(TOPK, T)
    te_arr = te.reshape(128)[:NTILES]

    xs = _sc_scatter(xbf, pos2)
    ys = _grouped(te_arr, xs,
                  expert_gate_W.astype(bf), expert_up_W.astype(bf),
                  expert_down_W.astype(bf))
    ycomb = _sc_gather(ys, pos2)

    shared_raw = _shared(xbf, shared_gate_W.astype(bf), shared_up_W.astype(bf),
                         shared_down_W.astype(bf))
    final = _combine(shared_raw, ycomb, w2, sig)
    return final.reshape(B, S, H), logits


# speedup vs baseline: 28.6419x; 28.6419x over previous
import jax
import jax.numpy as jnp
from jax.experimental import pallas as pl
from jax.experimental.pallas import tpu as pltpu
from jax.experimental.pallas import tpu_sc as plsc

B, S, H = 2, 4096, 1024
E, TOPK = 8, 2
INTER = 4096
MOE_INTER = INTER // TOPK
T = B * S

TT_R = 2048
RTILE = 256
NTILES = T * TOPK // RTILE + E
NP = NTILES * RTILE
TT_S = 2048
MM_S = 512
TT_C = 1024
W_SC = 128
HH = H // 2


def _router_kernel(x_ref, w9_ref, logits_ref, idx_ref, w2_ref, sig_ref):
    x = x_ref[...]
    w9 = w9_ref[...]
    l9 = jax.lax.dot_general(x, w9, (((1,), (1,)), ((), ())),
                             preferred_element_type=jnp.float32)
    logits = l9[:, :E]
    logits_ref[...] = logits
    sig_ref[...] = jax.nn.sigmoid(l9[:, E:E + 1])
    m = jnp.max(logits, axis=1, keepdims=True)
    p = jnp.exp(logits - m)
    p = p / jnp.sum(p, axis=1, keepdims=True)
    w1 = jnp.max(p, axis=1, keepdims=True)
    i1 = jnp.argmax(p, axis=1)[:, None]
    cols = jax.lax.broadcasted_iota(jnp.int32, (TT_R, E), 1)
    p2 = jnp.where(cols == i1, -jnp.inf, p)
    w2 = jnp.max(p2, axis=1, keepdims=True)
    i2 = jnp.argmax(p2, axis=1)[:, None]
    idx_ref[...] = jnp.concatenate([i1, i2], axis=1).astype(jnp.int32)
    w2_ref[...] = jnp.concatenate([w1, w2], axis=1)


def _router(x32, w9):
    return pl.pallas_call(
        _router_kernel,
        grid=(T // TT_R,),
        in_specs=[
            pl.BlockSpec((TT_R, H), lambda t: (t, 0)),
            pl.BlockSpec((E + 1, H), lambda t: (0, 0)),
        ],
        out_specs=[
            pl.BlockSpec((TT_R, E), lambda t: (t, 0)),
            pl.BlockSpec((TT_R, TOPK), lambda t: (t, 0)),
            pl.BlockSpec((TT_R, TOPK), lambda t: (t, 0)),
            pl.BlockSpec((TT_R, 1), lambda t: (t, 0)),
        ],
        out_shape=[
            jax.ShapeDtypeStruct((T, E), jnp.float32),
            jax.ShapeDtypeStruct((T, TOPK), jnp.int32),
            jax.ShapeDtypeStruct((T, TOPK), jnp.float32),
            jax.ShapeDtypeStruct((T, 1), jnp.float32),
        ],
    )(x32, w9)


def _meta_kernel(ids_ref, pos_ref, te_ref):
    ids = ids_ref[...]
    r = jax.lax.broadcasted_iota(jnp.int32, (128, 128), 0)
    c = jax.lax.broadcasted_iota(jnp.int32, (128, 128), 1)
    tincl = (r <= c).astype(jnp.bfloat16)
    aexcl = (c < r).astype(jnp.bfloat16)

    cnts = [jnp.sum((ids == e).astype(jnp.int32)) for e in range(E)]
    goff, tstart, ptiles = [], [], []
    off = jnp.int32(0)
    toff = jnp.int32(0)
    for e in range(E):
        goff.append(off)
        tstart.append(toff)
        pt = (cnts[e] + (RTILE - 1)) // RTILE
        ptiles.append(pt)
        off = off + pt * RTILE
        toff = toff + pt

    pos = jnp.zeros((128, 128), jnp.float32)
    for e in range(E):
        me = ids == e
        mef = me.astype(jnp.bfloat16)
        rowcs = jax.lax.dot_general(mef, tincl, (((1,), (0,)), ((), ())),
                                    preferred_element_type=jnp.float32)
        rowsum = rowcs[:, 127:128].astype(jnp.bfloat16)
        roff = jax.lax.dot_general(aexcl, rowsum, (((1,), (0,)), ((), ())),
                                   preferred_element_type=jnp.float32)
        rank1 = rowcs + roff
        pos = pos + jnp.where(me, rank1 - 1.0 + goff[e].astype(jnp.float32), 0.0)
    pos_ref[...] = pos.astype(jnp.int32)

    ti = jax.lax.broadcasted_iota(jnp.int32, (1, 128), 1)
    te = jnp.zeros((1, 128), jnp.int32)
    for e in range(E):
        te = te + (ti >= tstart[e] + ptiles[e]).astype(jnp.int32)
    te_ref[...] = jnp.minimum(te, E - 1)


def _meta(ids128):
    return pl.pallas_call(
        _meta_kernel,
        out_shape=[
            jax.ShapeDtypeStruct((128, 128), jnp.int32),
            jax.ShapeDtypeStruct((1, 128), jnp.int32),
        ],
    )(ids128)


RW = 32
NWIN = T // RW


def _pad_indices(pos2):
    p = pos2.reshape(TOPK, NWIN, RW)
    pad = jnp.zeros((TOPK, NWIN, W_SC - RW), jnp.int32)
    return jnp.concatenate([p, pad], axis=2).reshape(TOPK, NWIN * W_SC)


def _sc_scatter(x32, idxp):
    mesh = plsc.VectorSubcoreMesh(core_axis_name="core", subcore_axis_name="subcore")

    @pl.kernel(out_type=jax.ShapeDtypeStruct((NP, H), jnp.float32), mesh=mesh)
    def k(x_hbm, p_hbm, o_hbm):
        def body(x_vmem, p_vmem):
            pltpu.sync_copy(x_vmem, o_hbm.at[p_vmem.at[0, pl.ds(0, RW)]])

        pltpu.emit_pipeline(
            body,
            grid=(TOPK, NWIN),
            in_specs=[
                pl.BlockSpec((RW, H), lambda kk, i: (i, 0)),
                pl.BlockSpec((1, W_SC), lambda kk, i: (kk, i)),
            ],
            out_specs=[],
            core_axis_name=("core", "subcore"),
            dimension_semantics=(pltpu.PARALLEL, pltpu.PARALLEL),
        )(x_hbm, p_hbm)

    return k(x32, idxp)


def _sc_gather(ys, idxp):
    mesh = plsc.VectorSubcoreMesh(core_axis_name="core", subcore_axis_name="subcore")

    @pl.kernel(out_type=jax.ShapeDtypeStruct((TOPK * T, H), jnp.float32), mesh=mesh)
    def k(ys_hbm, p_hbm, o_hbm):
        def body(p_vmem, o_vmem):
            pltpu.sync_copy(ys_hbm.at[p_vmem.at[0, pl.ds(0, RW)]], o_vmem)

        pltpu.emit_pipeline(
            body,
            grid=(TOPK, NWIN),
            in_specs=[pl.BlockSpec((1, W_SC), lambda kk, i: (kk, i))],
            out_specs=[pl.BlockSpec((RW, H), lambda kk, i: (kk * NWIN + i, 0))],
            core_axis_name=("core", "subcore"),
            dimension_semantics=(pltpu.PARALLEL, pltpu.PARALLEL),
        )(p_hbm, o_hbm)

    return k(ys, idxp)


def _group_kernel(te_ref, xs_ref, wg_ref, wu_ref, wd_ref, ys_ref):
    del te_ref
    x = xs_ref[...].astype(jnp.bfloat16)
    g = jax.lax.dot_general(x, wg_ref[0], (((1,), (1,)), ((), ())),
                            preferred_element_type=jnp.float32)
    u = jax.lax.dot_general(x, wu_ref[0], (((1,), (1,)), ((), ())),
                            preferred_element_type=jnp.float32)
    h = (g * jax.nn.sigmoid(g) * u).astype(jnp.bfloat16)
    ys_ref[...] = jax.lax.dot_general(h, wd_ref[0], (((1,), (1,)), ((), ())),
                                      preferred_element_type=jnp.float32)


def _grouped(te_arr, xs, wg, wu, wd):
    grid_spec = pltpu.PrefetchScalarGridSpec(
        num_scalar_prefetch=1,
        grid=(NTILES,),
        in_specs=[
            pl.BlockSpec((RTILE, H), lambda i, te: (i, 0)),
            pl.BlockSpec((1, MOE_INTER, H), lambda i, te: (te[i], 0, 0)),
            pl.BlockSpec((1, MOE_INTER, H), lambda i, te: (te[i], 0, 0)),
            pl.BlockSpec((1, H, MOE_INTER), lambda i, te: (te[i], 0, 0)),
        ],
        out_specs=pl.BlockSpec((RTILE, H), lambda i, te: (i, 0)),
    )
    return pl.pallas_call(
        _group_kernel,
        grid_spec=grid_spec,
        out_shape=jax.ShapeDtypeStruct((NP, H), jnp.float32),
    )(te_arr, xs, wg, wu, wd)


def _shared_kernel(x_ref, wsg_ref, wsu_ref, wsd_ref, out_ref):
    m = pl.program_id(1)

    @pl.when(m == 0)
    def _():
        out_ref[...] = jnp.zeros_like(out_ref)

    x = x_ref[...]
    g = jax.lax.dot_general(x, wsg_ref[...], (((1,), (1,)), ((), ())),
                            preferred_element_type=jnp.float32)
    u = jax.lax.dot_general(x, wsu_ref[...], (((1,), (1,)), ((), ())),
                            preferred_element_type=jnp.float32)
    h = (g * jax.nn.sigmoid(g) * u).astype(jnp.bfloat16)
    out_ref[...] += jax.lax.dot_general(h, wsd_ref[...], (((1,), (1,)), ((), ())),
                                        preferred_element_type=jnp.float32)


def _shared(xbf, wsg, wsu, wsd):
    return pl.pallas_call(
        _shared_kernel,
        grid=(T // TT_S, INTER // MM_S),
        in_specs=[
            pl.BlockSpec((TT_S, H), lambda t, m: (t, 0)),
            pl.BlockSpec((MM_S, H), lambda t, m: (m, 0)),
            pl.BlockSpec((MM_S, H), lambda t, m: (m, 0)),
            pl.BlockSpec((H, MM_S), lambda t, m: (0, m)),
        ],
        out_specs=pl.BlockSpec((TT_S, H), lambda t, m: (t, 0)),
        out_shape=jax.ShapeDtypeStruct((T, H), jnp.float32),
    )(xbf, wsg, wsu, wsd)


def _combine_kernel(sh_ref, y0_ref, y1_ref, w_ref, sig_ref, out_ref):
    w = w_ref[...]
    out_ref[...] = (sig_ref[...] * sh_ref[...]
                    + w[:, 0:1] * y0_ref[...] + w[:, 1:2] * y1_ref[...])


def _combine(shared_raw, ycomb, w2, sig):
    return pl.pallas_call(
        _combine_kernel,
        grid=(T // TT_C,),
        in_specs=[
            pl.BlockSpec((TT_C, H), lambda t: (t, 0)),
            pl.BlockSpec((TT_C, H), lambda t: (t, 0)),
            pl.BlockSpec((TT_C, H), lambda t: (T // TT_C + t, 0)),
            pl.BlockSpec((TT_C, TOPK), lambda t: (t, 0)),
            pl.BlockSpec((TT_C, 1), lambda t: (t, 0)),
        ],
        out_specs=pl.BlockSpec((TT_C, H), lambda t: (t, 0)),
        out_shape=jax.ShapeDtypeStruct((T, H), jnp.float32),
    )(shared_raw, ycomb, ycomb, w2, sig)


def kernel(hidden_states, gate_W, expert_gate_W, expert_up_W, expert_down_W,
           shared_gate_W, shared_up_W, shared_down_W, shared_expert_gate_W):
    x32 = hidden_states.reshape(T, H)
    xbf = x32.astype(jnp.bfloat16)
    bf = jnp.bfloat16

    w9 = jnp.concatenate([gate_W, shared_expert_gate_W], axis=0)
    logits, idx2, w2, sig = _router(x32, w9)

    ids128 = idx2.T.reshape(128, 128)
    pos128, te = _meta(ids128)
    pos2 = pos128.reshape(TOPK, T)
    te_arr = te.reshape(128)[:NTILES]
    idxp = _pad_indices(pos2)

    xs = _sc_scatter(x32, idxp)
    ys = _grouped(te_arr, xs,
                  expert_gate_W.astype(bf), expert_up_W.astype(bf),
                  expert_down_W.astype(bf))
    ycomb = _sc_gather(ys, idxp)

    shared_raw = _shared(xbf, shared_gate_W.astype(bf), shared_up_W.astype(bf),
                         shared_down_W.astype(bf))
    final = _combine(shared_raw, ycomb, w2, sig)
    return final.reshape(B, S, H), logits


# speedup vs baseline: 31.8978x; 1.1137x over previous
import jax
import jax.numpy as jnp
from jax.experimental import pallas as pl
from jax.experimental.pallas import tpu as pltpu
from jax.experimental.pallas import tpu_sc as plsc

B, S, H = 2, 4096, 1024
E, TOPK = 8, 2
INTER = 4096
MOE_INTER = INTER // TOPK
T = B * S

TT_R = 2048
RTILE = 256
NTILES = T * TOPK // RTILE + E
NP = NTILES * RTILE
TT_S = 2048
MM_S = 512
TT_C = 1024
W_SC = 128
HH = H // 2


def _router_kernel(x_ref, w9_ref, logits_ref, idx_ref, w2_ref, sig_ref):
    x = x_ref[...]
    w9 = w9_ref[...]
    l9 = jax.lax.dot_general(x, w9, (((1,), (1,)), ((), ())),
                             preferred_element_type=jnp.float32)
    logits = l9[:, :E]
    logits_ref[...] = logits
    sig_ref[...] = jax.nn.sigmoid(l9[:, E:E + 1])
    m = jnp.max(logits, axis=1, keepdims=True)
    p = jnp.exp(logits - m)
    p = p / jnp.sum(p, axis=1, keepdims=True)
    w1 = jnp.max(p, axis=1, keepdims=True)
    i1 = jnp.argmax(p, axis=1)[:, None]
    cols = jax.lax.broadcasted_iota(jnp.int32, (TT_R, E), 1)
    p2 = jnp.where(cols == i1, -jnp.inf, p)
    w2 = jnp.max(p2, axis=1, keepdims=True)
    i2 = jnp.argmax(p2, axis=1)[:, None]
    idx_ref[...] = jnp.concatenate([i1, i2], axis=1).astype(jnp.int32)
    w2_ref[...] = jnp.concatenate([w1, w2], axis=1)


def _router(x32, w9):
    return pl.pallas_call(
        _router_kernel,
        grid=(T // TT_R,),
        in_specs=[
            pl.BlockSpec((TT_R, H), lambda t: (t, 0)),
            pl.BlockSpec((E + 1, H), lambda t: (0, 0)),
        ],
        out_specs=[
            pl.BlockSpec((TT_R, E), lambda t: (t, 0)),
            pl.BlockSpec((TT_R, TOPK), lambda t: (t, 0)),
            pl.BlockSpec((TT_R, TOPK), lambda t: (t, 0)),
            pl.BlockSpec((TT_R, 1), lambda t: (t, 0)),
        ],
        out_shape=[
            jax.ShapeDtypeStruct((T, E), jnp.float32),
            jax.ShapeDtypeStruct((T, TOPK), jnp.int32),
            jax.ShapeDtypeStruct((T, TOPK), jnp.float32),
            jax.ShapeDtypeStruct((T, 1), jnp.float32),
        ],
    )(x32, w9)


def _meta_kernel(ids_ref, pos_ref, te_ref):
    ids = ids_ref[...]
    r = jax.lax.broadcasted_iota(jnp.int32, (128, 128), 0)
    c = jax.lax.broadcasted_iota(jnp.int32, (128, 128), 1)
    tincl = (r <= c).astype(jnp.bfloat16)
    aexcl = (c < r).astype(jnp.bfloat16)

    cnts = [jnp.sum((ids == e).astype(jnp.int32)) for e in range(E)]
    goff, tstart, ptiles = [], [], []
    off = jnp.int32(0)
    toff = jnp.int32(0)
    for e in range(E):
        goff.append(off)
        tstart.append(toff)
        pt = (cnts[e] + (RTILE - 1)) // RTILE
        ptiles.append(pt)
        off = off + pt * RTILE
        toff = toff + pt

    pos = jnp.zeros((128, 128), jnp.float32)
    for e in range(E):
        me = ids == e
        mef = me.astype(jnp.bfloat16)
        rowcs = jax.lax.dot_general(mef, tincl, (((1,), (0,)), ((), ())),
                                    preferred_element_type=jnp.float32)
        rowsum = rowcs[:, 127:128].astype(jnp.bfloat16)
        roff = jax.lax.dot_general(aexcl, rowsum, (((1,), (0,)), ((), ())),
                                   preferred_element_type=jnp.float32)
        rank1 = rowcs + roff
        pos = pos + jnp.where(me, rank1 - 1.0 + goff[e].astype(jnp.float32), 0.0)
    pos_ref[...] = pos.astype(jnp.int32)

    ti = jax.lax.broadcasted_iota(jnp.int32, (1, 128), 1)
    te = jnp.zeros((1, 128), jnp.int32)
    for e in range(E):
        te = te + (ti >= tstart[e] + ptiles[e]).astype(jnp.int32)
    te_ref[...] = jnp.minimum(te, E - 1)


def _meta(ids128):
    return pl.pallas_call(
        _meta_kernel,
        out_shape=[
            jax.ShapeDtypeStruct((128, 128), jnp.int32),
            jax.ShapeDtypeStruct((1, 128), jnp.int32),
        ],
    )(ids128)


RW = 32
NWIN = T // RW


def _pad_indices(pos2):
    p = pos2.reshape(TOPK, NWIN, RW)
    pad = jnp.zeros((TOPK, NWIN, W_SC - RW), jnp.int32)
    return jnp.concatenate([p, pad], axis=2).reshape(TOPK, NWIN * W_SC)


def _sc_scatter(x32, idxp):
    mesh = plsc.VectorSubcoreMesh(core_axis_name="core", subcore_axis_name="subcore")

    @pl.kernel(out_type=jax.ShapeDtypeStruct((NP, H), jnp.float32), mesh=mesh)
    def k(x_hbm, p_hbm, o_hbm):
        def body(x_vmem, p_vmem):
            pltpu.sync_copy(x_vmem, o_hbm.at[p_vmem.at[0, pl.ds(0, RW)]])

        pltpu.emit_pipeline(
            body,
            grid=(TOPK, NWIN),
            in_specs=[
                pl.BlockSpec((RW, H), lambda kk, i: (i, 0)),
                pl.BlockSpec((1, W_SC), lambda kk, i: (kk, i)),
            ],
            out_specs=[],
            core_axis_name=("core", "subcore"),
            dimension_semantics=(pltpu.PARALLEL, pltpu.PARALLEL),
        )(x_hbm, p_hbm)

    return k(x32, idxp)


def _sc_gather(ys, idxp):
    mesh = plsc.VectorSubcoreMesh(core_axis_name="core", subcore_axis_name="subcore")

    @pl.kernel(out_type=jax.ShapeDtypeStruct((TOPK * T, H), jnp.float32), mesh=mesh)
    def k(ys_hbm, p_hbm, o_hbm):
        def body(p_vmem, o_vmem):
            pltpu.sync_copy(ys_hbm.at[p_vmem.at[0, pl.ds(0, RW)]], o_vmem)

        pltpu.emit_pipeline(
            body,
            grid=(TOPK, NWIN),
            in_specs=[pl.BlockSpec((1, W_SC), lambda kk, i: (kk, i))],
            out_specs=[pl.BlockSpec((RW, H), lambda kk, i: (kk * NWIN + i, 0))],
            core_axis_name=("core", "subcore"),
            dimension_semantics=(pltpu.PARALLEL, pltpu.PARALLEL),
        )(p_hbm, o_hbm)

    return k(ys, idxp)


MM_G = MOE_INTER // 2


def _group_kernel_a(te_ref, xs_ref, wg_ref, wu_ref, wd_ref, ys_ref):
    del te_ref
    x = xs_ref[...].astype(jnp.bfloat16)
    wg = wg_ref[0].astype(jnp.bfloat16)
    wu = wu_ref[0].astype(jnp.bfloat16)
    wd = wd_ref[0].astype(jnp.bfloat16)
    g = jax.lax.dot_general(x, wg, (((1,), (1,)), ((), ())),
                            preferred_element_type=jnp.float32)
    u = jax.lax.dot_general(x, wu, (((1,), (1,)), ((), ())),
                            preferred_element_type=jnp.float32)
    h = (g * jax.nn.sigmoid(g) * u).astype(jnp.bfloat16)
    ys_ref[...] = jax.lax.dot_general(h, wd, (((1,), (1,)), ((), ())),
                                      preferred_element_type=jnp.float32)


def _group_kernel_b(te_ref, xs_ref, wg_ref, wu_ref, wd_ref, prev_ref, ys_ref):
    del te_ref
    x = xs_ref[...].astype(jnp.bfloat16)
    wg = wg_ref[0].astype(jnp.bfloat16)
    wu = wu_ref[0].astype(jnp.bfloat16)
    wd = wd_ref[0].astype(jnp.bfloat16)
    g = jax.lax.dot_general(x, wg, (((1,), (1,)), ((), ())),
                            preferred_element_type=jnp.float32)
    u = jax.lax.dot_general(x, wu, (((1,), (1,)), ((), ())),
                            preferred_element_type=jnp.float32)
    h = (g * jax.nn.sigmoid(g) * u).astype(jnp.bfloat16)
    ys_ref[...] = prev_ref[...] + jax.lax.dot_general(
        h, wd, (((1,), (1,)), ((), ())), preferred_element_type=jnp.float32)


def _grouped_half(te_arr, xs, wg, wu, wd, m, prev):
    body = _group_kernel_a if prev is None else _group_kernel_b
    in_specs = [
        pl.BlockSpec((RTILE, H), lambda i, te: (i, 0)),
        pl.BlockSpec((1, MM_G, H), lambda i, te: (te[i], m, 0)),
        pl.BlockSpec((1, MM_G, H), lambda i, te: (te[i], m, 0)),
        pl.BlockSpec((1, H, MM_G), lambda i, te: (te[i], 0, m)),
    ]
    args = [te_arr, xs, wg, wu, wd]
    if prev is not None:
        in_specs.append(pl.BlockSpec((RTILE, H), lambda i, te: (i, 0)))
        args.append(prev)
    grid_spec = pltpu.PrefetchScalarGridSpec(
        num_scalar_prefetch=1,
        grid=(NTILES,),
        in_specs=in_specs,
        out_specs=pl.BlockSpec((RTILE, H), lambda i, te: (i, 0)),
    )
    return pl.pallas_call(
        body,
        grid_spec=grid_spec,
        out_shape=jax.ShapeDtypeStruct((NP, H), jnp.float32),
    )(*args)


def _grouped(te_arr, xs, wg, wu, wd):
    ys_a = _grouped_half(te_arr, xs, wg, wu, wd, 0, None)
    return _grouped_half(te_arr, xs, wg, wu, wd, 1, ys_a)


def _shared_kernel(x_ref, wsg_ref, wsu_ref, wsd_ref, out_ref):
    m = pl.program_id(1)

    @pl.when(m == 0)
    def _():
        out_ref[...] = jnp.zeros_like(out_ref)

    x = x_ref[...].astype(jnp.bfloat16)
    wsg = wsg_ref[...].astype(jnp.bfloat16)
    wsu = wsu_ref[...].astype(jnp.bfloat16)
    wsd = wsd_ref[...].astype(jnp.bfloat16)
    g = jax.lax.dot_general(x, wsg, (((1,), (1,)), ((), ())),
                            preferred_element_type=jnp.float32)
    u = jax.lax.dot_general(x, wsu, (((1,), (1,)), ((), ())),
                            preferred_element_type=jnp.float32)
    h = (g * jax.nn.sigmoid(g) * u).astype(jnp.bfloat16)
    out_ref[...] += jax.lax.dot_general(h, wsd, (((1,), (1,)), ((), ())),
                                        preferred_element_type=jnp.float32)


def _shared(xbf, wsg, wsu, wsd):
    return pl.pallas_call(
        _shared_kernel,
        grid=(T // TT_S, INTER // MM_S),
        in_specs=[
            pl.BlockSpec((TT_S, H), lambda t, m: (t, 0)),
            pl.BlockSpec((MM_S, H), lambda t, m: (m, 0)),
            pl.BlockSpec((MM_S, H), lambda t, m: (m, 0)),
            pl.BlockSpec((H, MM_S), lambda t, m: (0, m)),
        ],
        out_specs=pl.BlockSpec((TT_S, H), lambda t, m: (t, 0)),
        out_shape=jax.ShapeDtypeStruct((T, H), jnp.float32),
    )(xbf, wsg, wsu, wsd)


def _combine_kernel(sh_ref, y0_ref, y1_ref, w_ref, sig_ref, out_ref):
    w = w_ref[...]
    out_ref[...] = (sig_ref[...] * sh_ref[...]
                    + w[:, 0:1] * y0_ref[...] + w[:, 1:2] * y1_ref[...])


def _combine(shared_raw, ycomb, w2, sig):
    return pl.pallas_call(
        _combine_kernel,
        grid=(T // TT_C,),
        in_specs=[
            pl.BlockSpec((TT_C, H), lambda t: (t, 0)),
            pl.BlockSpec((TT_C, H), lambda t: (t, 0)),
            pl.BlockSpec((TT_C, H), lambda t: (T // TT_C + t, 0)),
            pl.BlockSpec((TT_C, TOPK), lambda t: (t, 0)),
            pl.BlockSpec((TT_C, 1), lambda t: (t, 0)),
        ],
        out_specs=pl.BlockSpec((TT_C, H), lambda t: (t, 0)),
        out_shape=jax.ShapeDtypeStruct((T, H), jnp.float32),
    )(shared_raw, ycomb, ycomb, w2, sig)


def kernel(hidden_states, gate_W, expert_gate_W, expert_up_W, expert_down_W,
           shared_gate_W, shared_up_W, shared_down_W, shared_expert_gate_W):
    x32 = hidden_states.reshape(T, H)

    w9 = jnp.concatenate([gate_W, shared_expert_gate_W], axis=0)
    logits, idx2, w2, sig = _router(x32, w9)

    ids128 = idx2.T.reshape(128, 128)
    pos128, te = _meta(ids128)
    pos2 = pos128.reshape(TOPK, T)
    te_arr = te.reshape(128)
    idxp = _pad_indices(pos2)

    shared_raw = _shared(x32, shared_gate_W, shared_up_W, shared_down_W)

    xs = _sc_scatter(x32, idxp)
    ys = _grouped(te_arr, xs, expert_gate_W, expert_up_W, expert_down_W)
    ycomb = _sc_gather(ys, idxp)

    final = _combine(shared_raw, ycomb, w2, sig)
    return final.reshape(B, S, H), logits
